# baseline (speedup 1.0000x reference)
import functools

import jax
import jax.numpy as jnp
from jax import lax
from jax.experimental import pallas as pl
from jax.experimental.pallas import tpu as pltpu
from jax.experimental.pallas import tpu_sc as plsc

B = 16384
ALPHA = 0.5
MARGIN = 1.0
NS = 16
L = 16
CHUNK = B // NS
STEPS = CHUNK // L

_mesh = plsc.VectorSubcoreMesh(
    core_axis_name="c", subcore_axis_name="s", num_cores=1, num_subcores=NS
)


@functools.partial(
    pl.kernel,
    out_type=jax.ShapeDtypeStruct((8,), jnp.float32),
    mesh=_mesh,
    scratch_types=[
        pltpu.VMEM((2 * CHUNK,), jnp.float32),
        pltpu.VMEM((CHUNK,), jnp.int32),
        pltpu.VMEM((CHUNK,), jnp.float32),
        pltpu.VMEM((CHUNK,), jnp.float32),
        pltpu.VMEM((2, L), jnp.float32),
        pltpu.VMEM((NS, 2, L), jnp.float32),
        pltpu.HBM((NS, 2, L), jnp.float32),
        pltpu.SemaphoreType.DMA,
    ],
    compiler_params=pltpu.CompilerParams(needs_layout_passes=False),
)
def _loss_kernel(inp_hbm, tgt_hbm, scan_hbm, diag_hbm, out_hbm,
                 inp_v, tgt_v, scan_v, diag_v, st_v, all_v, part_hbm, sem):
    sid = lax.axis_index("s")
    base = sid * CHUNK

    copies = [
        pltpu.async_copy(inp_hbm.at[pl.ds(2 * base, 2 * CHUNK)], inp_v, sem),
        pltpu.async_copy(tgt_hbm.at[pl.ds(base, CHUNK)], tgt_v, sem),
        pltpu.async_copy(scan_hbm.at[pl.ds(base, CHUNK)], scan_v, sem),
        pltpu.async_copy(diag_hbm.at[pl.ds(base, CHUNK)], diag_v, sem),
    ]
    for c in copies:
        c.wait()

    lane = lax.iota(jnp.int32, L)

    def body(i, accs):
        acc0, acc1 = accs
        off = i * L
        rows2 = (lane + off) * 2
        t = tgt_v[pl.ds(off, L)]
        sc = scan_v[pl.ds(off, L)]
        dg = diag_v[pl.ds(off, L)]

        picked = plsc.load_gather(inp_v, [rows2 + t])
        other = plsc.load_gather(inp_v, [rows2 + (1 - t)])

        s = other - picked
        u = jnp.exp(-jnp.abs(s))
        r = u / (u + 2.0)
        r2 = r * r
        p = 1.0 / 9.0
        p = p * r2 + 1.0 / 7.0
        p = p * r2 + 1.0 / 5.0
        p = p * r2 + 1.0 / 3.0
        p = p * r2 + 1.0
        ce = jnp.maximum(s, 0.0) + (2.0 * r) * p
        acc0 = acc0 + ce

        d0 = sc - dg
        pos = t > 0
        diff = jnp.where(pos, d0 + MARGIN, jnp.minimum(0.0, d0 - MARGIN))
        diff = jnp.where(pos & (dg < -MARGIN), jnp.maximum(0.0, sc + MARGIN), diff)
        acc1 = acc1 + diff * diff
        return acc0, acc1

    zero = jnp.zeros((L,), jnp.float32)
    acc0, acc1 = lax.fori_loop(0, STEPS, body, (zero, zero), unroll=4)

    st_v[0, :] = acc0
    st_v[1, :] = acc1
    pltpu.sync_copy(st_v, part_hbm.at[sid])
    plsc.subcore_barrier()

    @pl.when(sid == 0)
    def _():
        pltpu.sync_copy(part_hbm, all_v)
        tot0 = jnp.zeros((L,), jnp.float32)
        tot1 = jnp.zeros((L,), jnp.float32)
        for w in range(NS):
            tot0 = tot0 + all_v[w, 0]
            tot1 = tot1 + all_v[w, 1]
        it0 = jnp.sum(tot0) * (1.0 / B)
        it1 = jnp.sum(tot1) * (1.0 / B)
        loss = it0 + ALPHA * it1
        st_v[0, :] = jnp.where(
            lane == 0,
            jnp.full((L,), it1, jnp.float32),
            jnp.full((L,), loss, jnp.float32),
        )
        pltpu.sync_copy(st_v.at[0, pl.ds(0, 8)], out_hbm)


def kernel(inputs, targets, scan_t, diag_t):
    out = _loss_kernel(jnp.reshape(inputs, (-1,)), targets, scan_t, diag_t)
    return out[0], out[1]


# speedup vs baseline: 1.0274x; 1.0274x over previous
import functools

import jax
import jax.numpy as jnp
from jax import lax
from jax.experimental import pallas as pl
from jax.experimental.pallas import tpu as pltpu
from jax.experimental.pallas import tpu_sc as plsc

B = 16384
ALPHA = 0.5
MARGIN = 1.0
NS = 16
L = 16
CHUNK = B // NS
STEPS = CHUNK // L

_mesh = plsc.VectorSubcoreMesh(
    core_axis_name="c", subcore_axis_name="s", num_cores=1, num_subcores=NS
)


@functools.partial(
    pl.kernel,
    out_type=[
        jax.ShapeDtypeStruct((1,), jnp.float32),
        jax.ShapeDtypeStruct((1,), jnp.float32),
    ],
    mesh=_mesh,
    scratch_types=[
        pltpu.VMEM((2 * CHUNK,), jnp.float32),
        pltpu.VMEM((CHUNK,), jnp.int32),
        pltpu.VMEM((CHUNK,), jnp.float32),
        pltpu.VMEM((CHUNK,), jnp.float32),
        pltpu.VMEM((2, L), jnp.float32),
        pltpu.VMEM((NS, 2, L), jnp.float32),
        pltpu.HBM((NS, 2, L), jnp.float32),
        pltpu.SemaphoreType.DMA,
    ],
    compiler_params=pltpu.CompilerParams(needs_layout_passes=False),
)
def _loss_kernel(inp_hbm, tgt_hbm, scan_hbm, diag_hbm, o1_hbm, o2_hbm,
                 inp_v, tgt_v, scan_v, diag_v, st_v, all_v, part_hbm, sem):
    sid = lax.axis_index("s")
    base = sid * CHUNK

    copies = [
        pltpu.async_copy(inp_hbm.at[pl.ds(2 * base, 2 * CHUNK)], inp_v, sem),
        pltpu.async_copy(tgt_hbm.at[pl.ds(base, CHUNK)], tgt_v, sem),
        pltpu.async_copy(scan_hbm.at[pl.ds(base, CHUNK)], scan_v, sem),
        pltpu.async_copy(diag_hbm.at[pl.ds(base, CHUNK)], diag_v, sem),
    ]
    for c in copies:
        c.wait()

    lane = lax.iota(jnp.int32, L)

    def body(i, accs):
        acc0, acc1 = accs
        off = i * L
        rows2 = (lane + off) * 2
        t = tgt_v[pl.ds(off, L)]
        sc = scan_v[pl.ds(off, L)]
        dg = diag_v[pl.ds(off, L)]

        picked = plsc.load_gather(inp_v, [rows2 + t])
        other = plsc.load_gather(inp_v, [rows2 + (1 - t)])

        s = other - picked
        u = jnp.exp(-jnp.abs(s))
        r = u / (u + 2.0)
        r2 = r * r
        p = 1.0 / 9.0
        p = p * r2 + 1.0 / 7.0
        p = p * r2 + 1.0 / 5.0
        p = p * r2 + 1.0 / 3.0
        p = p * r2 + 1.0
        ce = jnp.maximum(s, 0.0) + (2.0 * r) * p
        acc0 = acc0 + ce

        d0 = sc - dg
        pos = t > 0
        diff = jnp.where(pos, d0 + MARGIN, jnp.minimum(0.0, d0 - MARGIN))
        diff = jnp.where(pos & (dg < -MARGIN), jnp.maximum(0.0, sc + MARGIN), diff)
        acc1 = acc1 + diff * diff
        return acc0, acc1

    zero = jnp.zeros((L,), jnp.float32)
    acc0, acc1 = lax.fori_loop(0, STEPS, body, (zero, zero), unroll=4)

    st_v[0, :] = acc0
    st_v[1, :] = acc1
    pltpu.sync_copy(st_v, part_hbm.at[sid])
    plsc.subcore_barrier()

    @pl.when(sid == 0)
    def _():
        pltpu.sync_copy(part_hbm, all_v)
        tot0 = jnp.zeros((L,), jnp.float32)
        tot1 = jnp.zeros((L,), jnp.float32)
        for w in range(NS):
            tot0 = tot0 + all_v[w, 0]
            tot1 = tot1 + all_v[w, 1]
        it0 = jnp.sum(tot0) * (1.0 / B)
        it1 = jnp.sum(tot1) * (1.0 / B)
        loss = it0 + ALPHA * it1
        st_v[0, :] = jnp.full((L,), it1, jnp.float32)
        st_v[1, :] = jnp.full((L,), loss, jnp.float32)
        pltpu.sync_copy(st_v.at[0, pl.ds(0, 1)], o1_hbm)
        pltpu.sync_copy(st_v.at[1, pl.ds(0, 1)], o2_hbm)


def kernel(inputs, targets, scan_t, diag_t):
    it1, loss = _loss_kernel(jnp.reshape(inputs, (-1,)), targets, scan_t, diag_t)
    return jnp.reshape(it1, ()), jnp.reshape(loss, ())


# speedup vs baseline: 1.0542x; 1.0260x over previous
import functools

import jax
import jax.numpy as jnp
from jax import lax
from jax.experimental import pallas as pl
from jax.experimental.pallas import tpu as pltpu
from jax.experimental.pallas import tpu_sc as plsc

B = 16384
ALPHA = 0.5
MARGIN = 1.0
NS = 16
L = 16
CHUNK = B // NS
STEPS = CHUNK // L

_mesh = plsc.VectorSubcoreMesh(
    core_axis_name="c", subcore_axis_name="s", num_cores=1, num_subcores=NS
)


@functools.partial(
    pl.kernel,
    out_type=[
        jax.ShapeDtypeStruct((1,), jnp.float32),
        jax.ShapeDtypeStruct((1,), jnp.float32),
    ],
    mesh=_mesh,
    scratch_types=[
        pltpu.VMEM((2 * CHUNK,), jnp.float32),
        pltpu.VMEM((CHUNK,), jnp.int32),
        pltpu.VMEM((CHUNK,), jnp.float32),
        pltpu.VMEM((CHUNK,), jnp.float32),
        pltpu.VMEM((2, L), jnp.float32),
        pltpu.SMEM((2,), jnp.int32),
        pltpu.SemaphoreType.DMA,
    ],
    compiler_params=pltpu.CompilerParams(needs_layout_passes=False),
)
def _loss_kernel(inp_hbm, tgt_hbm, scan_hbm, diag_hbm, o1_hbm, o2_hbm,
                 inp_v, tgt_v, scan_v, diag_v, st_v, acc_smem, sem):
    sid = lax.axis_index("s")
    base = sid * CHUNK

    @pl.when(sid == 0)
    def _():
        acc_smem[0] = 0
        acc_smem[1] = 0

    plsc.subcore_barrier()

    copies = [
        pltpu.async_copy(inp_hbm.at[pl.ds(2 * base, 2 * CHUNK)], inp_v, sem),
        pltpu.async_copy(tgt_hbm.at[pl.ds(base, CHUNK)], tgt_v, sem),
        pltpu.async_copy(scan_hbm.at[pl.ds(base, CHUNK)], scan_v, sem),
        pltpu.async_copy(diag_hbm.at[pl.ds(base, CHUNK)], diag_v, sem),
    ]
    for c in copies:
        c.wait()

    lane = lax.iota(jnp.int32, L)

    def body(i, accs):
        acc0, acc1 = accs
        off = i * L
        rows2 = (lane + off) * 2
        t = tgt_v[pl.ds(off, L)]
        sc = scan_v[pl.ds(off, L)]
        dg = diag_v[pl.ds(off, L)]

        picked = plsc.load_gather(inp_v, [rows2 + t])
        other = plsc.load_gather(inp_v, [rows2 + (1 - t)])

        s = other - picked
        u = jnp.exp(-jnp.abs(s))
        r = u / (u + 2.0)
        r2 = r * r
        p = 1.0 / 9.0
        p = p * r2 + 1.0 / 7.0
        p = p * r2 + 1.0 / 5.0
        p = p * r2 + 1.0 / 3.0
        p = p * r2 + 1.0
        ce = jnp.maximum(s, 0.0) + (2.0 * r) * p
        acc0 = acc0 + ce

        d0 = sc - dg
        pos = t > 0
        diff = jnp.where(pos, d0 + MARGIN, jnp.minimum(0.0, d0 - MARGIN))
        diff = jnp.where(pos & (dg < -MARGIN), jnp.maximum(0.0, sc + MARGIN), diff)
        acc1 = acc1 + diff * diff
        return acc0, acc1

    zero = jnp.zeros((L,), jnp.float32)
    acc0, acc1 = lax.fori_loop(0, STEPS, body, (zero, zero), unroll=4)

    SCALE = 256.0
    q0 = (jnp.sum(acc0) * SCALE + 0.5).astype(jnp.int32)
    q1 = (jnp.sum(acc1) * SCALE + 0.5).astype(jnp.int32)
    plsc.fetch_and_add(acc_smem.at[0], q0, subcore_id=0)
    plsc.fetch_and_add(acc_smem.at[1], q1, subcore_id=0)
    plsc.subcore_barrier()

    @pl.when(sid == 0)
    def _():
        it0 = acc_smem[0].astype(jnp.float32) * (1.0 / (SCALE * B))
        it1 = acc_smem[1].astype(jnp.float32) * (1.0 / (SCALE * B))
        loss = it0 + ALPHA * it1
        st_v[0, :] = jnp.full((L,), it1, jnp.float32)
        st_v[1, :] = jnp.full((L,), loss, jnp.float32)
        pltpu.sync_copy(st_v.at[0, pl.ds(0, 1)], o1_hbm)
        pltpu.sync_copy(st_v.at[1, pl.ds(0, 1)], o2_hbm)


def kernel(inputs, targets, scan_t, diag_t):
    it1, loss = _loss_kernel(jnp.reshape(inputs, (-1,)), targets, scan_t, diag_t)
    return jnp.reshape(it1, ()), jnp.reshape(loss, ())


# speedup vs baseline: 1.0569x; 1.0026x over previous
import functools

import jax
import jax.numpy as jnp
from jax import lax
from jax.experimental import pallas as pl
from jax.experimental.pallas import tpu as pltpu
from jax.experimental.pallas import tpu_sc as plsc

B = 16384
ALPHA = 0.5
MARGIN = 1.0
NS = 16
L = 16
CHUNK = B // NS
STEPS = CHUNK // L

_mesh = plsc.VectorSubcoreMesh(
    core_axis_name="c", subcore_axis_name="s", num_cores=1, num_subcores=NS
)


@functools.partial(
    pl.kernel,
    out_type=[
        jax.ShapeDtypeStruct((1,), jnp.float32),
        jax.ShapeDtypeStruct((1,), jnp.float32),
    ],
    mesh=_mesh,
    scratch_types=[
        pltpu.VMEM((2 * CHUNK,), jnp.float32),
        pltpu.VMEM((CHUNK,), jnp.int32),
        pltpu.VMEM((CHUNK,), jnp.float32),
        pltpu.VMEM((CHUNK,), jnp.float32),
        pltpu.VMEM((2, L), jnp.float32),
        pltpu.SMEM((2,), jnp.int32),
        pltpu.SemaphoreType.DMA,
    ],
    compiler_params=pltpu.CompilerParams(needs_layout_passes=False),
)
def _loss_kernel(inp_hbm, tgt_hbm, scan_hbm, diag_hbm, o1_hbm, o2_hbm,
                 inp_v, tgt_v, scan_v, diag_v, st_v, acc_smem, sem):
    sid = lax.axis_index("s")
    base = sid * CHUNK

    copies = [
        pltpu.async_copy(inp_hbm.at[pl.ds(2 * base, 2 * CHUNK)], inp_v, sem),
        pltpu.async_copy(tgt_hbm.at[pl.ds(base, CHUNK)], tgt_v, sem),
        pltpu.async_copy(scan_hbm.at[pl.ds(base, CHUNK)], scan_v, sem),
        pltpu.async_copy(diag_hbm.at[pl.ds(base, CHUNK)], diag_v, sem),
    ]

    @pl.when(sid == 0)
    def _():
        acc_smem[0] = 0
        acc_smem[1] = 0

    plsc.subcore_barrier()
    for c in copies:
        c.wait()

    lane = lax.iota(jnp.int32, L)

    def body(i, accs):
        acc0, acc1 = accs
        off = i * L
        rows2 = (lane + off) * 2
        t = tgt_v[pl.ds(off, L)]
        sc = scan_v[pl.ds(off, L)]
        dg = diag_v[pl.ds(off, L)]

        ip = rows2 + t
        picked = plsc.load_gather(inp_v, [ip])
        other = plsc.load_gather(inp_v, [ip ^ 1])

        s = other - picked
        u = jnp.exp(-jnp.abs(s))
        r = u / (u + 2.0)
        r2 = r * r
        p = 1.0 / 7.0
        p = p * r2 + 1.0 / 5.0
        p = p * r2 + 1.0 / 3.0
        p = p * r2 + 1.0
        ce = jnp.maximum(s, 0.0) + (2.0 * r) * p
        acc0 = acc0 + ce

        d0 = sc - dg
        pos = t > 0
        diff = jnp.where(pos, d0 + MARGIN, jnp.minimum(0.0, d0 - MARGIN))
        diff = jnp.where(pos & (dg < -MARGIN), jnp.maximum(0.0, sc + MARGIN), diff)
        acc1 = acc1 + diff * diff
        return acc0, acc1

    zero = jnp.zeros((L,), jnp.float32)
    acc0, acc1 = lax.fori_loop(0, STEPS, body, (zero, zero), unroll=8)

    SCALE = 256.0
    q0 = (jnp.sum(acc0) * SCALE + 0.5).astype(jnp.int32)
    q1 = (jnp.sum(acc1) * SCALE + 0.5).astype(jnp.int32)
    plsc.fetch_and_add(acc_smem.at[0], q0, subcore_id=0)
    plsc.fetch_and_add(acc_smem.at[1], q1, subcore_id=0)
    plsc.subcore_barrier()

    @pl.when(sid == 0)
    def _():
        it0 = acc_smem[0].astype(jnp.float32) * (1.0 / (SCALE * B))
        it1 = acc_smem[1].astype(jnp.float32) * (1.0 / (SCALE * B))
        loss = it0 + ALPHA * it1
        st_v[0, :] = jnp.full((L,), it1, jnp.float32)
        st_v[1, :] = jnp.full((L,), loss, jnp.float32)
        pltpu.sync_copy(st_v.at[0, pl.ds(0, 1)], o1_hbm)
        pltpu.sync_copy(st_v.at[1, pl.ds(0, 1)], o2_hbm)


def kernel(inputs, targets, scan_t, diag_t):
    it1, loss = _loss_kernel(jnp.reshape(inputs, (-1,)), targets, scan_t, diag_t)
    return jnp.reshape(it1, ()), jnp.reshape(loss, ())
